# baseline (speedup 1.0000x reference)
import jax
import jax.numpy as jnp
from jax import lax
from jax.experimental import pallas as pl
from jax.experimental.pallas import tpu as pltpu
from jax.experimental.pallas import tpu_sc as plsc

N_FINE = 50000
N_COARSE = 10000
D_FEAT = 512

NC, NS = 2, 16
NW = NC * NS

CW = 112
NCH_H = 14
HPASS = 2
CH_T = CW * NCH_H * HPASS
HPAD = NS * CH_T
NBINS = 10240
BSL = NBINS // NS

GCH = 80
NCHUNK = N_FINE // GCH
KMAX = -(-NCHUNK // NW)
NBUF = 3


def _body(feat_hbm, idx_hbm, out_hbm, inv_hbm,
          idx_all, hidx_v, ones_v, slc_v, scale_v, sbc_v, rows_v,
          hist_sh, psem, hsem, gsems, ssems, osems):
    cid = lax.axis_index("c")
    sid = lax.axis_index("s")
    wid = cid * NS + sid
    start_c = wid * 19 + jnp.minimum(wid, 17)
    k_w = jnp.where(wid < 17, KMAX, KMAX - 1)

    @pl.when(wid < 17)
    def _():
        pltpu.async_copy(idx_hbm.at[pl.ds(start_c * GCH, KMAX * GCH)],
                         idx_all, psem)

    @pl.when(wid >= 17)
    def _():
        pltpu.async_copy(idx_hbm.at[pl.ds(start_c * GCH, (KMAX - 1) * GCH)],
                         idx_all.at[pl.ds(0, (KMAX - 1) * GCH)], psem)

    def fill(i, _):
        ones_v[0, pl.ds(i * 16, 16)] = jnp.ones((16,), jnp.float32)
        return 0

    lax.fori_loop(0, CW // 16, fill, 0)

    def fill_z(i, _):
        slc_v[pl.ds(i * 16, 16)] = jnp.zeros((16,), jnp.float32)
        return 0

    lax.fori_loop(0, BSL // 16, fill_z, 0)
    pltpu.sync_copy(slc_v, hist_sh.at[pl.ds(sid * BSL, BSL)])
    plsc.subcore_barrier()

    def hist_pass(hbase, nfull):
        for j in range(nfull):
            pltpu.async_copy(idx_hbm.at[pl.ds(hbase + j * CW, CW)],
                             hidx_v.at[j], hsem)
        for j in range(nfull):
            pltpu.make_async_copy(idx_hbm.at[pl.ds(hbase + j * CW, CW)],
                                  hidx_v.at[j], hsem).wait()
        for j in range(nfull):
            pltpu.async_copy(ones_v.at[0], hist_sh.at[hidx_v.at[j]], hsem,
                             add=True)
        for j in range(nfull):
            pltpu.make_async_copy(ones_v.at[0], hist_sh.at[hidx_v.at[j]],
                                  hsem).wait()

    @pl.when(sid < NS - 1)
    def _():
        for p in range(HPASS):
            hist_pass(sid * CH_T + p * (NCH_H * CW), NCH_H)

    @pl.when(sid == NS - 1)
    def _():
        base15 = (NS - 1) * CH_T
        hist_pass(base15, NCH_H)
        hist_pass(base15 + NCH_H * CW, 12)
        tail = base15 + (NCH_H + 12) * CW
        pltpu.sync_copy(idx_hbm.at[pl.ds(tail, 48)],
                        hidx_v.at[13, pl.ds(0, 48)])
        for t in range(4):
            hidx_v[13, pl.ds(48 + t * 16, 16)] = jnp.full(
                (16,), N_COARSE, jnp.int32)
        pltpu.sync_copy(ones_v.at[0], hist_sh.at[hidx_v.at[13]], add=True)

    plsc.subcore_barrier()

    pltpu.sync_copy(hist_sh.at[pl.ds(sid * BSL, BSL)], slc_v)

    def inv_step(i, _):
        c = slc_v[pl.ds(i * 16, 16)]
        slc_v[pl.ds(i * 16, 16)] = 1.0 / jnp.maximum(c, 1.0)
        return 0

    lax.fori_loop(0, BSL // 16, inv_step, 0)
    pltpu.sync_copy(slc_v, inv_hbm.at[pl.ds(sid * BSL, BSL)])
    plsc.subcore_barrier()

    @pl.when(wid < 17)
    def _():
        pltpu.make_async_copy(idx_hbm.at[pl.ds(start_c * GCH, KMAX * GCH)],
                              idx_all, psem).wait()

    @pl.when(wid >= 17)
    def _():
        pltpu.make_async_copy(
            idx_hbm.at[pl.ds(start_c * GCH, (KMAX - 1) * GCH)],
            idx_all.at[pl.ds(0, (KMAX - 1) * GCH)], psem).wait()

    def idx_slice(k):
        return idx_all.at[pl.ds(k * GCH, GCH)]

    def start_gather(k, b):
        pltpu.async_copy(feat_hbm.at[idx_slice(k)], rows_v.at[b], gsems.at[b])
        pltpu.async_copy(inv_hbm.at[idx_slice(k)], scale_v.at[b],
                         ssems.at[b])

    def consume(k, b):
        pltpu.make_async_copy(feat_hbm.at[idx_slice(k)], rows_v.at[b],
                              gsems.at[b]).wait()
        pltpu.make_async_copy(inv_hbm.at[idx_slice(k)],
                              scale_v.at[b], ssems.at[b]).wait()
        for g in range(GCH // 16):
            sv = scale_v[b, pl.ds(g * 16, 16)]
            for l in range(16):
                sbc_v[pl.ds((g * 16 + l) * 16, 16)] = jnp.full(
                    (16,), sv[l], jnp.float32)

        def row(r, _):
            sb = sbc_v[pl.ds(r * 16, 16)]
            for i in range(D_FEAT // 16):
                rows_v[b, r, pl.ds(i * 16, 16)] = (
                    rows_v[b, r, pl.ds(i * 16, 16)] * sb)
            return 0

        lax.fori_loop(0, GCH, row, 0)
        pltpu.async_copy(rows_v.at[b],
                         out_hbm.at[pl.ds((start_c + k) * GCH, GCH)],
                         osems.at[b])

    def drain_scatter(b):
        pltpu.make_async_copy(rows_v.at[b], out_hbm.at[pl.ds(0, GCH)],
                              osems.at[b]).wait()

    start_gather(0, 0)
    start_gather(1, 1)

    @pl.loop(0, KMAX - 2, step=NBUF)
    def _(k):
        for d in range(NBUF):
            kk = k + d
            b = d
            b3 = (d + 2) % NBUF
            consume(kk, b)

            @pl.when(jnp.logical_and(kk >= 1, kk + 2 < k_w))
            def _():
                drain_scatter(b3)

            @pl.when(kk + 2 < k_w)
            def _():
                start_gather(kk + 2, b3)

    consume(KMAX - 2, 0)

    @pl.when(k_w == KMAX)
    def _():
        consume(KMAX - 1, 1)

    for b in range(NBUF):
        drain_scatter(b)


def _fused(features, idx_pad):
    k = pl.kernel(
        _body,
        out_type=(jax.ShapeDtypeStruct((N_FINE, D_FEAT), jnp.float32),
                  jax.ShapeDtypeStruct((NBINS,), jnp.float32)),
        mesh=plsc.VectorSubcoreMesh(core_axis_name="c", subcore_axis_name="s",
                                    num_cores=NC, num_subcores=NS),
        scratch_types=[
            pltpu.VMEM((KMAX * GCH,), jnp.int32),
            pltpu.VMEM((NCH_H, CW), jnp.int32),
            pltpu.VMEM((1, CW), jnp.float32),
            pltpu.VMEM((BSL,), jnp.float32),
            pltpu.VMEM((NBUF, GCH), jnp.float32),
            pltpu.VMEM((GCH * 16,), jnp.float32),
            pltpu.VMEM((NBUF, GCH, D_FEAT), jnp.float32),
            pltpu.VMEM_SHARED((NBINS,), jnp.float32),
            pltpu.SemaphoreType.DMA,
            pltpu.SemaphoreType.DMA,
            pltpu.SemaphoreType.DMA((NBUF,)),
            pltpu.SemaphoreType.DMA((NBUF,)),
            pltpu.SemaphoreType.DMA((NBUF,)),
        ],
    )
    out, _ = k(features, idx_pad)
    return out


def kernel(features, cluster):
    return _fused(features, cluster.astype(jnp.int32))


# speedup vs baseline: 1.0046x; 1.0046x over previous
import jax
import jax.numpy as jnp
from jax import lax
from jax.experimental import pallas as pl
from jax.experimental.pallas import tpu as pltpu
from jax.experimental.pallas import tpu_sc as plsc

N_FINE = 50000
N_COARSE = 10000
D_FEAT = 512

NC, NS = 2, 16
NW = NC * NS

CW = 112
NCH_H = 14
HPASS = 2
CH_T = CW * NCH_H * HPASS
HPAD = NS * CH_T
NBINS = 10240
BSL = NBINS // NS

GCH = 80
NCHUNK = N_FINE // GCH
KMAX = -(-NCHUNK // NW)
NBUF = 3


def _body(feat_hbm, idx_hbm, out_hbm, inv_hbm,
          idx_all, hidx_v, ones_v, slc_v, scale_v, sbc_v, rows_v,
          hist_sh, psem, hsem, gsems, ssems, osems):
    cid = lax.axis_index("c")
    sid = lax.axis_index("s")
    wid = cid * NS + sid
    start_c = wid * 19 + jnp.minimum(wid, 17)
    k_w = jnp.where(wid < 17, KMAX, KMAX - 1)

    @pl.when(wid < 17)
    def _():
        pltpu.async_copy(idx_hbm.at[pl.ds(start_c * GCH, KMAX * GCH)],
                         idx_all, psem)

    @pl.when(wid >= 17)
    def _():
        pltpu.async_copy(idx_hbm.at[pl.ds(start_c * GCH, (KMAX - 1) * GCH)],
                         idx_all.at[pl.ds(0, (KMAX - 1) * GCH)], psem)

    def fill(i, _):
        ones_v[0, pl.ds(i * 16, 16)] = jnp.ones((16,), jnp.float32)
        return 0

    lax.fori_loop(0, CW // 16, fill, 0)

    def fill_z(i, _):
        slc_v[pl.ds(i * 16, 16)] = jnp.zeros((16,), jnp.float32)
        return 0

    lax.fori_loop(0, BSL // 16, fill_z, 0)
    pltpu.sync_copy(slc_v, hist_sh.at[pl.ds(sid * BSL, BSL)])

    @pl.when(wid < 17)
    def _():
        pltpu.make_async_copy(idx_hbm.at[pl.ds(start_c * GCH, KMAX * GCH)],
                              idx_all, psem).wait()

    @pl.when(wid >= 17)
    def _():
        pltpu.make_async_copy(
            idx_hbm.at[pl.ds(start_c * GCH, (KMAX - 1) * GCH)],
            idx_all.at[pl.ds(0, (KMAX - 1) * GCH)], psem).wait()

    def _idx_slice(k):
        return idx_all.at[pl.ds(k * GCH, GCH)]

    for kb in range(2):
        pltpu.async_copy(feat_hbm.at[_idx_slice(kb)], rows_v.at[kb],
                         gsems.at[kb])

    plsc.subcore_barrier()

    def hist_pass(hbase, nfull):
        for j in range(nfull):
            pltpu.async_copy(idx_hbm.at[pl.ds(hbase + j * CW, CW)],
                             hidx_v.at[j], hsem)
        for j in range(nfull):
            pltpu.make_async_copy(idx_hbm.at[pl.ds(hbase + j * CW, CW)],
                                  hidx_v.at[j], hsem).wait()
        for j in range(nfull):
            pltpu.async_copy(ones_v.at[0], hist_sh.at[hidx_v.at[j]], hsem,
                             add=True)
        for j in range(nfull):
            pltpu.make_async_copy(ones_v.at[0], hist_sh.at[hidx_v.at[j]],
                                  hsem).wait()

    @pl.when(sid < NS - 1)
    def _():
        for p in range(HPASS):
            hist_pass(sid * CH_T + p * (NCH_H * CW), NCH_H)

    @pl.when(sid == NS - 1)
    def _():
        base15 = (NS - 1) * CH_T
        hist_pass(base15, NCH_H)
        hist_pass(base15 + NCH_H * CW, 12)
        tail = base15 + (NCH_H + 12) * CW
        pltpu.sync_copy(idx_hbm.at[pl.ds(tail, 48)],
                        hidx_v.at[13, pl.ds(0, 48)])
        for t in range(4):
            hidx_v[13, pl.ds(48 + t * 16, 16)] = jnp.full(
                (16,), N_COARSE, jnp.int32)
        pltpu.sync_copy(ones_v.at[0], hist_sh.at[hidx_v.at[13]], add=True)

    plsc.subcore_barrier()

    pltpu.sync_copy(hist_sh.at[pl.ds(sid * BSL, BSL)], slc_v)

    def inv_step(i, _):
        c = slc_v[pl.ds(i * 16, 16)]
        slc_v[pl.ds(i * 16, 16)] = 1.0 / jnp.maximum(c, 1.0)
        return 0

    lax.fori_loop(0, BSL // 16, inv_step, 0)
    pltpu.sync_copy(slc_v, inv_hbm.at[pl.ds(sid * BSL, BSL)])
    plsc.subcore_barrier()

    idx_slice = _idx_slice

    def start_gather(k, b):
        pltpu.async_copy(feat_hbm.at[idx_slice(k)], rows_v.at[b], gsems.at[b])
        pltpu.async_copy(inv_hbm.at[idx_slice(k)], scale_v.at[b],
                         ssems.at[b])

    def start_scale(k, b):
        pltpu.async_copy(inv_hbm.at[idx_slice(k)], scale_v.at[b],
                         ssems.at[b])

    def consume(k, b):
        pltpu.make_async_copy(feat_hbm.at[idx_slice(k)], rows_v.at[b],
                              gsems.at[b]).wait()
        pltpu.make_async_copy(inv_hbm.at[idx_slice(k)],
                              scale_v.at[b], ssems.at[b]).wait()
        for g in range(GCH // 16):
            sv = scale_v[b, pl.ds(g * 16, 16)]
            for l in range(16):
                sbc_v[pl.ds((g * 16 + l) * 16, 16)] = jnp.full(
                    (16,), sv[l], jnp.float32)

        def row(r, _):
            sb = sbc_v[pl.ds(r * 16, 16)]
            for i in range(D_FEAT // 16):
                rows_v[b, r, pl.ds(i * 16, 16)] = (
                    rows_v[b, r, pl.ds(i * 16, 16)] * sb)
            return 0

        lax.fori_loop(0, GCH, row, 0)
        pltpu.async_copy(rows_v.at[b],
                         out_hbm.at[pl.ds((start_c + k) * GCH, GCH)],
                         osems.at[b])

    def drain_scatter(b):
        pltpu.make_async_copy(rows_v.at[b], out_hbm.at[pl.ds(0, GCH)],
                              osems.at[b]).wait()

    start_scale(0, 0)
    start_scale(1, 1)

    @pl.loop(0, KMAX - 2, step=NBUF)
    def _(k):
        for d in range(NBUF):
            kk = k + d
            b = d
            b3 = (d + 2) % NBUF
            consume(kk, b)

            @pl.when(jnp.logical_and(kk >= 1, kk + 2 < k_w))
            def _():
                drain_scatter(b3)

            @pl.when(kk + 2 < k_w)
            def _():
                start_gather(kk + 2, b3)

    consume(KMAX - 2, 0)

    @pl.when(k_w == KMAX)
    def _():
        consume(KMAX - 1, 1)

    for b in range(NBUF):
        drain_scatter(b)


def _fused(features, idx_pad):
    k = pl.kernel(
        _body,
        out_type=(jax.ShapeDtypeStruct((N_FINE, D_FEAT), jnp.float32),
                  jax.ShapeDtypeStruct((NBINS,), jnp.float32)),
        mesh=plsc.VectorSubcoreMesh(core_axis_name="c", subcore_axis_name="s",
                                    num_cores=NC, num_subcores=NS),
        scratch_types=[
            pltpu.VMEM((KMAX * GCH,), jnp.int32),
            pltpu.VMEM((NCH_H, CW), jnp.int32),
            pltpu.VMEM((1, CW), jnp.float32),
            pltpu.VMEM((BSL,), jnp.float32),
            pltpu.VMEM((NBUF, GCH), jnp.float32),
            pltpu.VMEM((GCH * 16,), jnp.float32),
            pltpu.VMEM((NBUF, GCH, D_FEAT), jnp.float32),
            pltpu.VMEM_SHARED((NBINS,), jnp.float32),
            pltpu.SemaphoreType.DMA,
            pltpu.SemaphoreType.DMA,
            pltpu.SemaphoreType.DMA((NBUF,)),
            pltpu.SemaphoreType.DMA((NBUF,)),
            pltpu.SemaphoreType.DMA((NBUF,)),
        ],
    )
    out, _ = k(features, idx_pad)
    return out


def kernel(features, cluster):
    return _fused(features, cluster.astype(jnp.int32))
